# baseline (speedup 1.0000x reference)
import jax
import jax.numpy as jnp
from jax.experimental import pallas as pl
from jax.experimental.pallas import tpu as pltpu


def _convtr_kernel(xs_ref, w8_ref, b_ref, o_ref):
    c_in3, q = xs_ref.shape
    c_in = c_in3 // 3
    s_hi = xs_ref[0:2 * c_in, :]
    s_lo = xs_ref[c_in:3 * c_in, :]
    dn = (((0,), (0,)), ((), ()))
    bias = b_ref[...]
    for p in range(8):
        s = s_lo if p < 4 else s_hi
        o_ref[p] = jax.lax.dot_general(
            w8_ref[p], s, dn, preferred_element_type=jnp.float32) + bias


def kernel(v, g, bias, x):
    c_in, c_out, k = v.shape
    n, _, l_in = x.shape
    s, pad = 8, 4
    l_out = (l_in - 1) * s - 2 * pad + k
    q_len = -(-l_out // s)

    norm = jnp.sqrt(jnp.sum(v * v, axis=(1, 2), keepdims=True))
    w = (g * v / norm)

    w8 = jnp.stack(
        [jnp.concatenate([w[:, :, p + 4], w[:, :, p + 12]], axis=0)
         for p in range(4)] +
        [jnp.concatenate([w[:, :, p - 4], w[:, :, p + 4]], axis=0)
         for p in range(4, 8)],
        axis=0).astype(jnp.bfloat16)

    xpad = jnp.pad(x, ((0, 0), (0, 0), (1, q_len + 1 - l_in)))
    xs = jnp.concatenate(
        [xpad[:, :, 2:q_len + 2], xpad[:, :, 1:q_len + 1], xpad[:, :, 0:q_len]],
        axis=1).astype(jnp.bfloat16)

    bias_col = bias.astype(jnp.float32)[:, None]

    out = pl.pallas_call(
        _convtr_kernel,
        out_shape=jax.ShapeDtypeStruct((n, 8, c_out, q_len), jnp.float32),
        grid=(n,),
        in_specs=[
            pl.BlockSpec((None, 3 * c_in, q_len), lambda b: (b, 0, 0)),
            pl.BlockSpec((8, 2 * c_in, c_out), lambda b: (0, 0, 0)),
            pl.BlockSpec((c_out, 1), lambda b: (0, 0)),
        ],
        out_specs=pl.BlockSpec((None, 8, c_out, q_len), lambda b: (b, 0, 0, 0)),
        compiler_params=pltpu.CompilerParams(
            dimension_semantics=("parallel",)),
    )(xs, w8, bias_col)

    out = jnp.transpose(out, (0, 2, 3, 1)).reshape(n, c_out, q_len * s)
    return out[:, :, :l_out]
